# Initial kernel scaffold; baseline (speedup 1.0000x reference)
import jax
import jax.numpy as jnp
from jax.experimental import pallas as pl


def kernel(input_, weight):
    raise NotImplementedError("write your pallas kernel here")


# speedup vs baseline: 1.8761x; 1.8761x over previous
import functools

import jax
import jax.numpy as jnp
from jax import lax
from jax.experimental import pallas as pl
from jax.experimental.pallas import tpu as pltpu
from jax.experimental.pallas import tpu_sc as plsc

_BATCH = 16384
_SEQ = 50
_DIM = 64
_N = _BATCH * _SEQ
_NC = 2
_NS = 16
_NW = _NC * _NS
_PER_W = _N // _NW
_CHUNK = 128
_NCH = _PER_W // _CHUNK
_NB = 4

_mesh = plsc.VectorSubcoreMesh(core_axis_name="c", subcore_axis_name="s")


@functools.partial(
    pl.kernel,
    out_type=jax.ShapeDtypeStruct((_N, _DIM), jnp.float32),
    mesh=_mesh,
    compiler_params=pltpu.CompilerParams(use_tc_tiling_on_sc=False),
    scratch_types=[
        pltpu.VMEM((_NCH, _CHUNK), jnp.int32),
        pltpu.VMEM((_NB, _CHUNK, _DIM), jnp.float32),
        [pltpu.SemaphoreType.DMA] * _NB,
    ],
)
def _embedding_gather(idx_hbm, table_hbm, out_hbm, idx_v, rows_v, sems):
    wid = lax.axis_index("s") * _NC + lax.axis_index("c")
    base_chunk = wid * _NCH
    base_row = wid * _PER_W

    pltpu.sync_copy(idx_hbm.at[pl.ds(base_chunk, _NCH)], idx_v)

    for b in range(_NB):
        pltpu.async_copy(table_hbm.at[idx_v.at[b]], rows_v.at[b], sems[b])

    def body(g, carry):
        for b in range(_NB):
            j = g * _NB + b
            pltpu.make_async_copy(
                table_hbm.at[idx_v.at[b]], rows_v.at[b], sems[b]
            ).wait()
            pltpu.sync_copy(
                rows_v.at[b], out_hbm.at[pl.ds(base_row + j * _CHUNK, _CHUNK)]
            )

            @pl.when(j + _NB < _NCH)
            def _():
                pltpu.async_copy(
                    table_hbm.at[idx_v.at[j + _NB]], rows_v.at[b], sems[b]
                )

        return carry

    lax.fori_loop(0, _NCH // _NB, body, 0, unroll=False)


def kernel(input_, weight):
    idx = input_.astype(jnp.int32).reshape(_NW * _NCH, _CHUNK)
    out = _embedding_gather(idx, weight)
    return out.reshape(_BATCH, _SEQ, _DIM)
